# baseline (speedup 1.0000x reference)
import jax
import jax.numpy as jnp
from jax import lax
from jax.experimental import pallas as pl
from jax.experimental.pallas import tpu as pltpu
from jax.experimental.pallas import tpu_sc as plsc

N = 100000
E = 1600000
H = 64
BN = 2048
BE = 2048
EP = 1638400
GEP = EP // BE
PADE = EP - E
NC, NS = 2, 16
NW = NC * NS
EPW = EP // NW
KA = 2048
EPT = EP // NS
KC = 512
NPAD = 100352
SPT = NPAD // NS
GNP = NPAD // BN
ZW = 1568
ZR2 = 392

_F32 = jnp.float32


def _lrelu(v):
    return jnp.maximum(v, 0.01 * v)


def _sigm(v):
    return 1.0 / (1.0 + jnp.exp(-v))


def _elu(v):
    return jnp.where(v > 0, v, jnp.exp(jnp.minimum(v, 0.0)) - 1.0)


def _x_of(na_ref, w1r, b1r):
    return _lrelu(na_ref[...] * w1r[...] + b1r[...])


def _gru_new(h, x, WihT, WhhT, bihr, bhhr):
    gi = jnp.dot(h, WihT[...], preferred_element_type=_F32) + bihr[...]
    gh = jnp.dot(x, WhhT[...], preferred_element_type=_F32) + bhhr[...]
    rg = _sigm(gi[:, 0:H] + gh[:, 0:H])
    zg = _sigm(gi[:, H:2 * H] + gh[:, H:2 * H])
    ng = jnp.tanh(gi[:, 2 * H:3 * H] + rg * gh[:, 2 * H:3 * H])
    return (1.0 - zg) * ng + zg * x


def _xnew(hn0, hn1, hn2, hn3, ssum, na, W2T, gbr, WihT, WhhT, bihr, bhhr,
          w1r, b1r):
    hnum = jnp.concatenate(
        [hn0[...], hn1[...], hn2[...], hn3[...]], axis=1)
    hpre = hnum / (ssum[...] + 1e-16)
    h = _elu(jnp.dot(hpre, W2T[...], preferred_element_type=_F32) + gbr[...])
    x = _x_of(na, w1r, b1r)
    return jnp.maximum(_gru_new(h, x, WihT, WhhT, bihr, bhhr), 0.0)


def _mask_col(i):
    return (lax.broadcasted_iota(jnp.int32, (BN, 1), 0) + i * BN) < N


def _pre_body(na, w1r, b1r, attrr, u_out):
    x = _x_of(na, w1r, b1r)
    u_out[...] = jnp.sum(x * attrr[...], axis=1, keepdims=True)


def _gather_body(na, u, src, dst, asrc, udst, idx_v, val_v, sem):
    c = lax.axis_index("c")
    s = lax.axis_index("s")
    wid = s * NC + c
    base = wid * EPW

    def chunk(i, carry):
        off = base + i * KA
        pltpu.sync_copy(src.at[pl.ds(off, KA)], idx_v)
        pltpu.async_copy(na.at[idx_v], val_v, sem).wait()
        pltpu.sync_copy(val_v, asrc.at[pl.ds(off, KA)])
        pltpu.sync_copy(dst.at[pl.ds(off, KA)], idx_v)
        pltpu.async_copy(u.at[idx_v], val_v, sem).wait()
        pltpu.sync_copy(val_v, udst.at[pl.ds(off, KA)])
        return carry

    lax.fori_loop(0, EPW // KA, chunk, 0)


def _edge_body(asrc, udst, t, w1r, b1r, W1aT, wer, attlr,
               e_out, m0, m1, m2, m3):
    x = _x_of(asrc, w1r, b1r)
    cc = jnp.dot(x, W1aT[...], preferred_element_type=_F32)
    m = _lrelu(cc + t[...] * wer[...])
    sv = jnp.sum(m * attlr[...], axis=1, keepdims=True)
    alpha = _lrelu(sv + udst[...])
    ev = jnp.exp(alpha)
    e_out[...] = ev
    msg = m * ev
    outs = (m0, m1, m2, m3)
    for g in range(4):
        outs[g][...] = msg[:, g * 16:(g + 1) * 16]


def _scatter_body(msg0, msg1, msg2, msg3, e, dst,
                  hn0, hn1, hn2, hn3, ss,
                  acc, sacc, data_v, idx_v, e_v, zb2, zb1, sem):
    del sem
    c = lax.axis_index("c")
    s = lax.axis_index("s")
    msgs = (msg0, msg1, msg2, msg3)
    hns = (hn0, hn1, hn2, hn3)
    zeros16 = jnp.zeros((16,), _F32)

    def zrow(i, carry):
        zb2[i, :] = zeros16
        return carry

    lax.fori_loop(0, ZR2, zrow, 0)

    def zrow1(i, carry):
        zb1[pl.ds(i * 16, 16)] = zeros16
        return carry

    lax.fori_loop(0, ZW // 16, zrow1, 0)

    for core in range(NC):
        @pl.when(c == core)
        def _(core=core):
            for r in range(2):
                g = core * 2 + r

                def zacc(j, carry):
                    pltpu.sync_copy(zb2, acc.at[pl.ds(s * SPT + j * ZR2, ZR2)])
                    return carry

                lax.fori_loop(0, SPT // ZR2, zacc, 0)
                if g == 0:
                    def zsacc(j, carry):
                        pltpu.sync_copy(
                            zb1, sacc.at[pl.ds(s * SPT + j * ZW, ZW)])
                        return carry
                    lax.fori_loop(0, SPT // ZW, zsacc, 0)
                plsc.subcore_barrier()

                def chunk(i, carry, g=g):
                    eb = s * EPT + i * KC
                    pltpu.sync_copy(dst.at[pl.ds(eb, KC)], idx_v)
                    pltpu.sync_copy(msgs[g].at[pl.ds(eb, KC)], data_v)
                    pltpu.sync_copy(data_v, acc.at[idx_v], add=True)
                    if g == 0:
                        pltpu.sync_copy(e.at[pl.ds(eb, KC)], e_v)
                        pltpu.sync_copy(e_v, sacc.at[idx_v], add=True)
                    return carry

                lax.fori_loop(0, EPT // KC, chunk, 0)
                plsc.subcore_barrier()
                pltpu.sync_copy(acc.at[pl.ds(s * SPT, SPT)],
                                hns[g].at[pl.ds(s * SPT, SPT)])
                if g == 0:
                    pltpu.sync_copy(sacc.at[pl.ds(s * SPT, SPT)],
                                    ss.at[pl.ds(s * SPT, SPT)])
                plsc.subcore_barrier()


def _nodesum_body(hn0, hn1, hn2, hn3, ssum, na, W2T, gbr, WihT, WhhT,
                  bihr, bhhr, w1r, b1r, sumx):
    i = pl.program_id(0)
    xn = _xnew(hn0, hn1, hn2, hn3, ssum, na, W2T, gbr, WihT, WhhT,
               bihr, bhhr, w1r, b1r)
    xn = jnp.where(_mask_col(i), xn, 0.0)

    @pl.when(i == 0)
    def _():
        sumx[...] = jnp.zeros_like(sumx)

    sumx[...] += jnp.sum(xn, axis=0, keepdims=True)


def _readout_body(hn0, hn1, hn2, hn3, ssum, na, W2T, gbr, WihT, WhhT,
                  bihr, bhhr, w1r, b1r, sumx, molWT, attsr, attdr, num, den):
    i = pl.program_id(0)
    xn = _xnew(hn0, hn1, hn2, hn3, ssum, na, W2T, gbr, WihT, WhhT,
               bihr, bhhr, w1r, b1r)
    xs = jnp.dot(xn, molWT[...], preferred_element_type=_F32)
    out0 = jnp.maximum(sumx[...], 0.0)
    xd = jnp.dot(out0, molWT[...], preferred_element_type=_F32)
    const = jnp.sum(xd * attdr[...])
    ap = _lrelu(jnp.sum(xs * attsr[...], axis=1, keepdims=True) + const)
    w = jnp.where(_mask_col(i), jnp.exp(ap), 0.0)

    @pl.when(i == 0)
    def _():
        num[...] = jnp.zeros_like(num)
        den[...] = jnp.zeros_like(den)

    num[...] += jnp.sum(xs * w, axis=0, keepdims=True)
    den[...] += jnp.sum(w).reshape(1, 1)


def _final_body(sumx, num, den, molbr, mWihT, mWhhT, mbihr, mbhhr,
                l2WT, l2br, out):
    out0 = jnp.maximum(sumx[...], 0.0)
    g = _elu(num[...] / den[0, 0] + molbr[...])
    o = jnp.maximum(_gru_new(g, out0, mWihT, mWhhT, mbihr, mbhhr), 0.0)
    out[...] = jnp.dot(o, l2WT[...], preferred_element_type=_F32) + l2br[...]


def _wspec(shape):
    return pl.BlockSpec(shape, lambda *_: tuple(0 for _ in shape))


_N2 = pl.BlockSpec((BN, 1), lambda i: (i, 0))
_E2 = pl.BlockSpec((BE, 1), lambda i: (i, 0))


def kernel(node_attr, edge_index, edge_attr, lin1_W, lin1_b, gate_lin1_W,
           gate_lin2_W, gate_att_l, gate_att_r, gate_bias, gru_W_ih, gru_W_hh,
           gru_b_ih, gru_b_hh, mol_W, mol_att_src, mol_att_dst, mol_bias,
           mgru_W_ih, mgru_W_hh, mgru_b_ih, mgru_b_hh, lin2_W, lin2_b):
    f32 = _F32
    na = jnp.concatenate([node_attr.reshape(N),
                          jnp.zeros((NPAD - N,), jnp.float32)])
    izeros = jnp.zeros((PADE,), jnp.int32)
    src = jnp.concatenate([edge_index[0], izeros])
    dst = jnp.concatenate([edge_index[1],
                           N + (jnp.arange(PADE, dtype=jnp.int32)
                                % (NPAD - N))])
    t_pad = jnp.concatenate([edge_attr.reshape(E),
                             jnp.zeros((PADE,), jnp.float32)])

    w1r = lin1_W.reshape(1, H)
    b1r = lin1_b.reshape(1, H)
    W1aT = gate_lin1_W[:, :H].T
    wer = gate_lin1_W[:, H].reshape(1, H)
    attlr = gate_att_l.reshape(1, H)
    attrr = gate_att_r.reshape(1, H)
    gbr = gate_bias.reshape(1, H)
    W2T = gate_lin2_W.T
    WihT = gru_W_ih.T
    WhhT = gru_W_hh.T
    bihr = gru_b_ih.reshape(1, 3 * H)
    bhhr = gru_b_hh.reshape(1, 3 * H)
    molWT = mol_W.T
    attsr = mol_att_src.reshape(1, H)
    attdr = mol_att_dst.reshape(1, H)
    molbr = mol_bias.reshape(1, H)
    mWihT = mgru_W_ih.T
    mWhhT = mgru_W_hh.T
    mbihr = mgru_b_ih.reshape(1, 3 * H)
    mbhhr = mgru_b_hh.reshape(1, 3 * H)
    l2WT = lin2_W.T
    l2br = lin2_b.reshape(1, H)

    u2 = pl.pallas_call(
        _pre_body,
        grid=(GNP,),
        in_specs=[_N2, _wspec((1, H)), _wspec((1, H)), _wspec((1, H))],
        out_specs=_N2,
        out_shape=jax.ShapeDtypeStruct((NPAD, 1), f32),
    )(na.reshape(NPAD, 1), w1r, b1r, attrr)

    asrc, udst = pl.kernel(
        _gather_body,
        out_type=(jax.ShapeDtypeStruct((EP,), f32),
                  jax.ShapeDtypeStruct((EP,), f32)),
        mesh=plsc.VectorSubcoreMesh(core_axis_name="c", subcore_axis_name="s"),
        scratch_types=(pltpu.VMEM((KA,), jnp.int32),
                       pltpu.VMEM((KA,), f32),
                       pltpu.SemaphoreType.DMA),
    )(na, u2.reshape(NPAD), src, dst)

    e2, m0, m1, m2, m3 = pl.pallas_call(
        _edge_body,
        grid=(GEP,),
        in_specs=[_E2, _E2, _E2,
                  _wspec((1, H)), _wspec((1, H)), _wspec((H, H)),
                  _wspec((1, H)), _wspec((1, H))],
        out_specs=[_E2] + [pl.BlockSpec((BE, 16), lambda i: (i, 0))] * 4,
        out_shape=[jax.ShapeDtypeStruct((EP, 1), f32)] +
                  [jax.ShapeDtypeStruct((EP, 16), f32)] * 4,
    )(asrc.reshape(EP, 1), udst.reshape(EP, 1), t_pad.reshape(EP, 1),
      w1r, b1r, W1aT, wer, attlr)

    hn0, hn1, hn2, hn3, ssum_pad = pl.kernel(
        _scatter_body,
        out_type=tuple([jax.ShapeDtypeStruct((NPAD, 16), f32)] * 4) +
                 (jax.ShapeDtypeStruct((NPAD,), f32),),
        mesh=plsc.VectorSubcoreMesh(core_axis_name="c", subcore_axis_name="s"),
        compiler_params=pltpu.CompilerParams(use_tc_tiling_on_sc=False),
        scratch_types=(pltpu.VMEM_SHARED((NPAD, 16), f32),
                       pltpu.VMEM_SHARED((NPAD,), f32),
                       pltpu.VMEM((KC, 16), f32),
                       pltpu.VMEM((KC,), jnp.int32),
                       pltpu.VMEM((KC,), f32),
                       pltpu.VMEM((ZR2, 16), f32),
                       pltpu.VMEM((ZW,), f32),
                       pltpu.SemaphoreType.DMA),
    )(m0, m1, m2, m3, e2.reshape(EP), dst)

    hnspec = [pl.BlockSpec((BN, 16), lambda i: (i, 0))] * 4
    node_ins = [hn0, hn1, hn2, hn3, ssum_pad.reshape(NPAD, 1),
                na.reshape(NPAD, 1), W2T, gbr, WihT, WhhT, bihr, bhhr,
                w1r, b1r]
    node_specs = hnspec + [_N2, _N2, _wspec((H, H)), _wspec((1, H)),
                           _wspec((H, 3 * H)), _wspec((H, 3 * H)),
                           _wspec((1, 3 * H)), _wspec((1, 3 * H)),
                           _wspec((1, H)), _wspec((1, H))]

    sumx = pl.pallas_call(
        _nodesum_body,
        grid=(GNP,),
        in_specs=node_specs,
        out_specs=pl.BlockSpec((1, H), lambda i: (0, 0)),
        out_shape=jax.ShapeDtypeStruct((1, H), f32),
    )(*node_ins)

    num, den = pl.pallas_call(
        _readout_body,
        grid=(GNP,),
        in_specs=node_specs + [_wspec((1, H)), _wspec((H, H)),
                               _wspec((1, H)), _wspec((1, H))],
        out_specs=[pl.BlockSpec((1, H), lambda i: (0, 0)),
                   pl.BlockSpec((1, 1), lambda i: (0, 0))],
        out_shape=[jax.ShapeDtypeStruct((1, H), f32),
                   jax.ShapeDtypeStruct((1, 1), f32)],
    )(*node_ins, sumx, molWT, attsr, attdr)

    res = pl.pallas_call(
        _final_body,
        in_specs=[_wspec((1, H)), _wspec((1, H)), _wspec((1, 1)),
                  _wspec((1, H)), _wspec((H, 3 * H)), _wspec((H, 3 * H)),
                  _wspec((1, 3 * H)), _wspec((1, 3 * H)),
                  _wspec((H, H)), _wspec((1, H))],
        out_specs=_wspec((1, H)),
        out_shape=jax.ShapeDtypeStruct((1, H), f32),
    )(sumx, num, den, molbr, mWihT, mWhhT, mbihr, mbhhr, l2WT, l2br)
    return res


# speedup vs baseline: 1.3341x; 1.3341x over previous
import jax
import jax.numpy as jnp
from jax import lax
from jax.experimental import pallas as pl
from jax.experimental.pallas import tpu as pltpu
from jax.experimental.pallas import tpu_sc as plsc

N = 100000
E = 1600000
H = 64
BN = 2048
BE = 2048
EP = 1638400
GEP = EP // BE
PADE = EP - E
NC, NS = 2, 16
NW = NC * NS
EPW = EP // NW
KA = 2048
EPT = EP // NS
KC = 512
NPAD = 100352
SPT = NPAD // NS
GNP = NPAD // BN
ZW = 1568
ZR2 = 392

_F32 = jnp.float32


def _lrelu(v):
    return jnp.maximum(v, 0.01 * v)


def _sigm(v):
    return 1.0 / (1.0 + jnp.exp(-v))


def _elu(v):
    return jnp.where(v > 0, v, jnp.exp(jnp.minimum(v, 0.0)) - 1.0)


def _col(row, ones11):
    return lax.dot_general(row, ones11[...], (((0,), (0,)), ((), ())),
                           preferred_element_type=_F32)


def _outer(row, wrow):
    return lax.dot_general(row, wrow, (((0,), (0,)), ((), ())),
                           preferred_element_type=_F32)


def _gru_new(h, x, WihT, WhhT, bihr, bhhr):
    gi = jnp.dot(h, WihT[...], preferred_element_type=_F32) + bihr[...]
    gh = jnp.dot(x, WhhT[...], preferred_element_type=_F32) + bhhr[...]
    rg = _sigm(gi[:, 0:H] + gh[:, 0:H])
    zg = _sigm(gi[:, H:2 * H] + gh[:, H:2 * H])
    ng = jnp.tanh(gi[:, 2 * H:3 * H] + rg * gh[:, 2 * H:3 * H])
    return (1.0 - zg) * ng + zg * x


def _xnew(hn0, hn1, hn2, hn3, ssum, na, W2T, gbr, Ws, w1r, b1r, ones11):
    hnum = jnp.concatenate(
        [hn0[...], hn1[...], hn2[...], hn3[...]], axis=1)
    scol = _col(ssum[0], ones11)
    hpre = hnum / (scol + 1e-16)
    h = _elu(jnp.dot(hpre, W2T[...], preferred_element_type=_F32) + gbr[...])
    x = _lrelu(_outer(na[0], w1r[...]) + b1r[...])
    Wr, Wz, Wn, Vr, Vz, Vn, bir, biz, bin_, bhr, bhz, bhn = Ws
    gr = jnp.dot(h, Wr[...], preferred_element_type=_F32) + bir[...]
    gz = jnp.dot(h, Wz[...], preferred_element_type=_F32) + biz[...]
    gn = jnp.dot(h, Wn[...], preferred_element_type=_F32) + bin_[...]
    hr = jnp.dot(x, Vr[...], preferred_element_type=_F32) + bhr[...]
    hz = jnp.dot(x, Vz[...], preferred_element_type=_F32) + bhz[...]
    hn_ = jnp.dot(x, Vn[...], preferred_element_type=_F32) + bhn[...]
    rg = _sigm(gr + hr)
    zg = _sigm(gz + hz)
    ng = jnp.tanh(gn + rg * hn_)
    return jnp.maximum((1.0 - zg) * ng + zg * x, 0.0)


def _mask_col(i):
    return (lax.broadcasted_iota(jnp.int32, (BN, 1), 0) + i * BN) < N


def _pre_body(na, w1c, b1c, attrc, u_out):
    xT = _lrelu(w1c[...] * na[0, 0][None, :] + b1c[...])
    u_out[...] = jnp.sum(xT * attrc[...], axis=0).reshape(1, 1, BN)


def _gather_body(na, u, src, dst, asrc, udst, idx_v, val_v, sem):
    c = lax.axis_index("c")
    s = lax.axis_index("s")
    wid = s * NC + c
    base = wid * EPW

    def chunk(i, carry):
        off = base + i * KA
        pltpu.sync_copy(src.at[pl.ds(off, KA)], idx_v)
        pltpu.async_copy(na.at[idx_v], val_v, sem).wait()
        pltpu.sync_copy(val_v, asrc.at[pl.ds(off, KA)])
        pltpu.sync_copy(dst.at[pl.ds(off, KA)], idx_v)
        pltpu.async_copy(u.at[idx_v], val_v, sem).wait()
        pltpu.sync_copy(val_v, udst.at[pl.ds(off, KA)])
        return carry

    lax.fori_loop(0, EPW // KA, chunk, 0)


def _edge_body(asrc, udst, t, w1c, b1c, W1a, wec, attlr, eye, ones11,
               e_out, m0, m1, m2, m3):
    a = asrc[0, 0]
    xT = _lrelu(w1c[...] * a[None, :] + b1c[...])
    cT = jnp.dot(W1a[...], xT, preferred_element_type=_F32)
    mT = _lrelu(cT + wec[...] * t[0, 0][None, :])
    sv = jnp.dot(attlr[...], mT, preferred_element_type=_F32)
    alpha = _lrelu(sv + udst[0])
    ev = jnp.exp(alpha)
    e_out[...] = ev.reshape(1, 1, BE)
    mE = lax.dot_general(mT, eye[...], (((0,), (0,)), ((), ())),
                         preferred_element_type=_F32)
    msg = mE * _col(ev, ones11)
    outs = (m0, m1, m2, m3)
    for g in range(4):
        outs[g][...] = msg[:, g * 16:(g + 1) * 16]


def _scatter_body(msg0, msg1, msg2, msg3, e, dst,
                  hn0, hn1, hn2, hn3, ss,
                  acc, sacc, data_v, idx_v, e_v, zb2, zb1, sem):
    del sem
    c = lax.axis_index("c")
    s = lax.axis_index("s")
    msgs = (msg0, msg1, msg2, msg3)
    hns = (hn0, hn1, hn2, hn3)
    zeros16 = jnp.zeros((16,), _F32)

    def zrow(i, carry):
        zb2[i, :] = zeros16
        return carry

    lax.fori_loop(0, ZR2, zrow, 0)

    def zrow1(i, carry):
        zb1[pl.ds(i * 16, 16)] = zeros16
        return carry

    lax.fori_loop(0, ZW // 16, zrow1, 0)

    for core in range(NC):
        @pl.when(c == core)
        def _(core=core):
            for r in range(2):
                g = core * 2 + r

                def zacc(j, carry):
                    pltpu.sync_copy(zb2, acc.at[pl.ds(s * SPT + j * ZR2, ZR2)])
                    return carry

                lax.fori_loop(0, SPT // ZR2, zacc, 0)
                if g == 0:
                    def zsacc(j, carry):
                        pltpu.sync_copy(
                            zb1, sacc.at[pl.ds(s * SPT + j * ZW, ZW)])
                        return carry
                    lax.fori_loop(0, SPT // ZW, zsacc, 0)
                plsc.subcore_barrier()

                def chunk(i, carry, g=g):
                    eb = s * EPT + i * KC
                    pltpu.sync_copy(dst.at[pl.ds(eb, KC)], idx_v)
                    pltpu.sync_copy(msgs[g].at[pl.ds(eb, KC)], data_v)
                    pltpu.sync_copy(data_v, acc.at[idx_v], add=True)
                    if g == 0:
                        pltpu.sync_copy(e.at[pl.ds(eb, KC)], e_v)
                        pltpu.sync_copy(e_v, sacc.at[idx_v], add=True)
                    return carry

                lax.fori_loop(0, EPT // KC, chunk, 0)
                plsc.subcore_barrier()
                pltpu.sync_copy(acc.at[pl.ds(s * SPT, SPT)],
                                hns[g].at[pl.ds(s * SPT, SPT)])
                if g == 0:
                    pltpu.sync_copy(sacc.at[pl.ds(s * SPT, SPT)],
                                    ss.at[pl.ds(s * SPT, SPT)])
                plsc.subcore_barrier()


def _nodesum_body(hn0, hn1, hn2, hn3, ssum, na, W2T, gbr,
                  Wr, Wz, Wn, Vr, Vz, Vn, bir, biz, bin_, bhr, bhz, bhn,
                  w1r, b1r, ones11, sumx):
    i = pl.program_id(0)
    Ws = (Wr, Wz, Wn, Vr, Vz, Vn, bir, biz, bin_, bhr, bhz, bhn)
    xn = _xnew(hn0, hn1, hn2, hn3, ssum, na, W2T, gbr, Ws, w1r, b1r, ones11)
    xn = jnp.where(_mask_col(i), xn, 0.0)

    @pl.when(i == 0)
    def _():
        sumx[...] = jnp.zeros_like(sumx)

    sumx[...] += jnp.sum(xn, axis=0, keepdims=True)


def _readout_body(hn0, hn1, hn2, hn3, ssum, na, W2T, gbr,
                  Wr, Wz, Wn, Vr, Vz, Vn, bir, biz, bin_, bhr, bhz, bhn,
                  w1r, b1r, ones11, sumx, molWT, attsc, attdr, num, den):
    i = pl.program_id(0)
    Ws = (Wr, Wz, Wn, Vr, Vz, Vn, bir, biz, bin_, bhr, bhz, bhn)
    xn = _xnew(hn0, hn1, hn2, hn3, ssum, na, W2T, gbr, Ws, w1r, b1r, ones11)
    xs = jnp.dot(xn, molWT[...], preferred_element_type=_F32)
    out0 = jnp.maximum(sumx[...], 0.0)
    xd = jnp.dot(out0, molWT[...], preferred_element_type=_F32)
    const = jnp.sum(xd * attdr[...])
    ap = _lrelu(jnp.dot(xs, attsc[...], preferred_element_type=_F32) + const)
    w = jnp.where(_mask_col(i), jnp.exp(ap), 0.0)

    @pl.when(i == 0)
    def _():
        num[...] = jnp.zeros_like(num)
        den[...] = jnp.zeros_like(den)

    num[...] += jnp.sum(xs * w, axis=0, keepdims=True)
    den[...] += jnp.sum(w).reshape(1, 1)


def _final_body(sumx, num, den, molbr, mWihT, mWhhT, mbihr, mbhhr,
                l2WT, l2br, out):
    out0 = jnp.maximum(sumx[...], 0.0)
    g = _elu(num[...] / den[0, 0] + molbr[...])
    o = jnp.maximum(_gru_new(g, out0, mWihT, mWhhT, mbihr, mbhhr), 0.0)
    out[...] = jnp.dot(o, l2WT[...], preferred_element_type=_F32) + l2br[...]


def _wspec(shape):
    return pl.BlockSpec(shape, lambda *_: tuple(0 for _ in shape))


_N3 = pl.BlockSpec((1, 1, BN), lambda i: (i, 0, 0))
_E3 = pl.BlockSpec((1, 1, BE), lambda i: (i, 0, 0))


def kernel(node_attr, edge_index, edge_attr, lin1_W, lin1_b, gate_lin1_W,
           gate_lin2_W, gate_att_l, gate_att_r, gate_bias, gru_W_ih, gru_W_hh,
           gru_b_ih, gru_b_hh, mol_W, mol_att_src, mol_att_dst, mol_bias,
           mgru_W_ih, mgru_W_hh, mgru_b_ih, mgru_b_hh, lin2_W, lin2_b):
    f32 = _F32
    na = jnp.concatenate([node_attr.reshape(N),
                          jnp.zeros((NPAD - N,), jnp.float32)])
    izeros = jnp.zeros((PADE,), jnp.int32)
    src = jnp.concatenate([edge_index[0], izeros])
    dst = jnp.concatenate([edge_index[1],
                           N + (jnp.arange(PADE, dtype=jnp.int32)
                                % (NPAD - N))])
    t_pad = jnp.concatenate([edge_attr.reshape(E),
                             jnp.zeros((PADE,), jnp.float32)])

    w1r = lin1_W.reshape(1, H)
    b1r = lin1_b.reshape(1, H)
    w1c = lin1_W.reshape(H, 1)
    b1c = lin1_b.reshape(H, 1)
    W1a = gate_lin1_W[:, :H]
    wec = gate_lin1_W[:, H].reshape(H, 1)
    attlr = gate_att_l.reshape(1, H)
    attrc = gate_att_r.reshape(H, 1)
    gbr = gate_bias.reshape(1, H)
    W2T = gate_lin2_W.T
    Wr, Wz, Wn = (gru_W_ih[i * H:(i + 1) * H].T for i in range(3))
    Vr, Vz, Vn = (gru_W_hh[i * H:(i + 1) * H].T for i in range(3))
    bir, biz, bin_ = (gru_b_ih[i * H:(i + 1) * H].reshape(1, H)
                      for i in range(3))
    bhr, bhz, bhn = (gru_b_hh[i * H:(i + 1) * H].reshape(1, H)
                     for i in range(3))
    molWT = mol_W.T
    attsc = mol_att_src.reshape(H, 1)
    attdr = mol_att_dst.reshape(1, H)
    molbr = mol_bias.reshape(1, H)
    mWihT = mgru_W_ih.T
    mWhhT = mgru_W_hh.T
    mbihr = mgru_b_ih.reshape(1, 3 * H)
    mbhhr = mgru_b_hh.reshape(1, 3 * H)
    l2WT = lin2_W.T
    l2br = lin2_b.reshape(1, H)
    eye = jnp.eye(H, dtype=f32)
    ones11 = jnp.ones((1, 1), f32)

    u3 = pl.pallas_call(
        _pre_body,
        grid=(GNP,),
        in_specs=[_N3, _wspec((H, 1)), _wspec((H, 1)), _wspec((H, 1))],
        out_specs=_N3,
        out_shape=jax.ShapeDtypeStruct((GNP, 1, BN), f32),
    )(na.reshape(GNP, 1, BN), w1c, b1c, attrc)

    asrc, udst = pl.kernel(
        _gather_body,
        out_type=(jax.ShapeDtypeStruct((EP,), f32),
                  jax.ShapeDtypeStruct((EP,), f32)),
        mesh=plsc.VectorSubcoreMesh(core_axis_name="c", subcore_axis_name="s"),
        scratch_types=(pltpu.VMEM((KA,), jnp.int32),
                       pltpu.VMEM((KA,), f32),
                       pltpu.SemaphoreType.DMA),
    )(na, u3.reshape(NPAD), src, dst)

    e3, m0, m1, m2, m3 = pl.pallas_call(
        _edge_body,
        grid=(GEP,),
        in_specs=[_E3, _E3, _E3,
                  _wspec((H, 1)), _wspec((H, 1)), _wspec((H, H)),
                  _wspec((H, 1)), _wspec((1, H)), _wspec((H, H)),
                  _wspec((1, 1))],
        out_specs=[_E3] + [pl.BlockSpec((BE, 16), lambda i: (i, 0))] * 4,
        out_shape=[jax.ShapeDtypeStruct((GEP, 1, BE), f32)] +
                  [jax.ShapeDtypeStruct((EP, 16), f32)] * 4,
    )(asrc.reshape(GEP, 1, BE), udst.reshape(GEP, 1, BE),
      t_pad.reshape(GEP, 1, BE), w1c, b1c, W1a, wec, attlr, eye, ones11)

    hn0, hn1, hn2, hn3, ssum_pad = pl.kernel(
        _scatter_body,
        out_type=tuple([jax.ShapeDtypeStruct((NPAD, 16), f32)] * 4) +
                 (jax.ShapeDtypeStruct((NPAD,), f32),),
        mesh=plsc.VectorSubcoreMesh(core_axis_name="c", subcore_axis_name="s"),
        compiler_params=pltpu.CompilerParams(use_tc_tiling_on_sc=False),
        scratch_types=(pltpu.VMEM_SHARED((NPAD, 16), f32),
                       pltpu.VMEM_SHARED((NPAD,), f32),
                       pltpu.VMEM((KC, 16), f32),
                       pltpu.VMEM((KC,), jnp.int32),
                       pltpu.VMEM((KC,), f32),
                       pltpu.VMEM((ZR2, 16), f32),
                       pltpu.VMEM((ZW,), f32),
                       pltpu.SemaphoreType.DMA),
    )(m0, m1, m2, m3, e3.reshape(EP), dst)

    hnspec = [pl.BlockSpec((BN, 16), lambda i: (i, 0))] * 4
    node_ins = [hn0, hn1, hn2, hn3, ssum_pad.reshape(GNP, 1, BN),
                na.reshape(GNP, 1, BN), W2T, gbr,
                Wr, Wz, Wn, Vr, Vz, Vn, bir, biz, bin_, bhr, bhz, bhn,
                w1r, b1r, ones11]
    node_specs = hnspec + [_N3, _N3, _wspec((H, H)), _wspec((1, H))] + \
                 [_wspec((H, H))] * 6 + [_wspec((1, H))] * 6 + \
                 [_wspec((1, H)), _wspec((1, H)), _wspec((1, 1))]

    sumx = pl.pallas_call(
        _nodesum_body,
        grid=(GNP,),
        in_specs=node_specs,
        out_specs=pl.BlockSpec((1, H), lambda i: (0, 0)),
        out_shape=jax.ShapeDtypeStruct((1, H), f32),
    )(*node_ins)

    num, den = pl.pallas_call(
        _readout_body,
        grid=(GNP,),
        in_specs=node_specs + [_wspec((1, H)), _wspec((H, H)),
                               _wspec((H, 1)), _wspec((1, H))],
        out_specs=[pl.BlockSpec((1, H), lambda i: (0, 0)),
                   pl.BlockSpec((1, 1), lambda i: (0, 0))],
        out_shape=[jax.ShapeDtypeStruct((1, H), f32),
                   jax.ShapeDtypeStruct((1, 1), f32)],
    )(*node_ins, sumx, molWT, attsc, attdr)

    res = pl.pallas_call(
        _final_body,
        in_specs=[_wspec((1, H)), _wspec((1, H)), _wspec((1, 1)),
                  _wspec((1, H)), _wspec((H, 3 * H)), _wspec((H, 3 * H)),
                  _wspec((1, 3 * H)), _wspec((1, 3 * H)),
                  _wspec((H, H)), _wspec((1, H))],
        out_specs=_wspec((1, H)),
        out_shape=jax.ShapeDtypeStruct((1, H), f32),
    )(sumx, num, den, molbr, mWihT, mWhhT, mbihr, mbhhr, l2WT, l2br)
    return res


# speedup vs baseline: 2.4850x; 1.8627x over previous
import jax
import jax.numpy as jnp
from jax import lax
from jax.experimental import pallas as pl
from jax.experimental.pallas import tpu as pltpu
from jax.experimental.pallas import tpu_sc as plsc

N = 100000
E = 1600000
H = 64
BN = 2048
BE = 2048
EP = 1638400
GEP = EP // BE
PADE = EP - E
NC, NS = 2, 16
NW = NC * NS
EPW = EP // NW
KA = 2048
EPT = EP // NS
KC = 512
NPAD = 100352
SPT = NPAD // NS
GNP = NPAD // BN
ZW = 1568
ZR2 = 196

_F32 = jnp.float32


def _lrelu(v):
    return jnp.maximum(v, 0.01 * v)


def _sigm(v):
    return 1.0 / (1.0 + jnp.exp(-v))


def _elu(v):
    return jnp.where(v > 0, v, jnp.exp(jnp.minimum(v, 0.0)) - 1.0)


def _col(row, ones11):
    return lax.dot_general(row, ones11[...], (((0,), (0,)), ((), ())),
                           preferred_element_type=_F32)


def _outer(row, wrow):
    return lax.dot_general(row, wrow, (((0,), (0,)), ((), ())),
                           preferred_element_type=_F32)


def _gru_new(h, x, WihT, WhhT, bihr, bhhr):
    gi = jnp.dot(h, WihT[...], preferred_element_type=_F32) + bihr[...]
    gh = jnp.dot(x, WhhT[...], preferred_element_type=_F32) + bhhr[...]
    rg = _sigm(gi[:, 0:H] + gh[:, 0:H])
    zg = _sigm(gi[:, H:2 * H] + gh[:, H:2 * H])
    ng = jnp.tanh(gi[:, 2 * H:3 * H] + rg * gh[:, 2 * H:3 * H])
    return (1.0 - zg) * ng + zg * x


def _xnew(hn0, hn1, hn2, hn3, ssum, na, W2T, gbr, Ws, w1r, b1r, ones11):
    hnum = jnp.concatenate(
        [hn0[...], hn1[...], hn2[...], hn3[...]], axis=1)
    scol = _col(ssum[0], ones11)
    hpre = hnum / (scol + 1e-16)
    h = _elu(jnp.dot(hpre, W2T[...], preferred_element_type=_F32) + gbr[...])
    x = _lrelu(_outer(na[0], w1r[...]) + b1r[...])
    Wr, Wz, Wn, Vr, Vz, Vn, bir, biz, bin_, bhr, bhz, bhn = Ws
    gr = jnp.dot(h, Wr[...], preferred_element_type=_F32) + bir[...]
    gz = jnp.dot(h, Wz[...], preferred_element_type=_F32) + biz[...]
    gn = jnp.dot(h, Wn[...], preferred_element_type=_F32) + bin_[...]
    hr = jnp.dot(x, Vr[...], preferred_element_type=_F32) + bhr[...]
    hz = jnp.dot(x, Vz[...], preferred_element_type=_F32) + bhz[...]
    hn_ = jnp.dot(x, Vn[...], preferred_element_type=_F32) + bhn[...]
    rg = _sigm(gr + hr)
    zg = _sigm(gz + hz)
    ng = jnp.tanh(gn + rg * hn_)
    return jnp.maximum((1.0 - zg) * ng + zg * x, 0.0)


def _mask_col(i):
    return (lax.broadcasted_iota(jnp.int32, (BN, 1), 0) + i * BN) < N


def _pre_body(na, w1c, b1c, attrc, u_out):
    xT = _lrelu(w1c[...] * na[0, 0][None, :] + b1c[...])
    u_out[...] = jnp.sum(xT * attrc[...], axis=0).reshape(1, 1, BN)


def _gather_body(na, u, src, dst, asrc, udst, idx_v, val_v, sem):
    c = lax.axis_index("c")
    s = lax.axis_index("s")
    wid = s * NC + c
    base = wid * EPW

    def chunk(i, carry):
        off = base + i * KA
        pltpu.sync_copy(src.at[pl.ds(off, KA)], idx_v)
        pltpu.async_copy(na.at[idx_v], val_v, sem).wait()
        pltpu.sync_copy(val_v, asrc.at[pl.ds(off, KA)])
        pltpu.sync_copy(dst.at[pl.ds(off, KA)], idx_v)
        pltpu.async_copy(u.at[idx_v], val_v, sem).wait()
        pltpu.sync_copy(val_v, udst.at[pl.ds(off, KA)])
        return carry

    lax.fori_loop(0, EPW // KA, chunk, 0)


def _edge_body(asrc, udst, t, w1c, b1c, W1a, wec, attlr, eye, ones11,
               e_out, m_out):
    a = asrc[0, 0]
    xT = _lrelu(w1c[...] * a[None, :] + b1c[...])
    cT = jnp.dot(W1a[...], xT, preferred_element_type=_F32)
    mT = _lrelu(cT + wec[...] * t[0, 0][None, :])
    sv = jnp.dot(attlr[...], mT, preferred_element_type=_F32)
    alpha = _lrelu(sv + udst[0])
    ev = jnp.exp(alpha)
    e_out[...] = ev.reshape(1, 1, BE)
    mE = lax.dot_general(mT, eye[...], (((0,), (0,)), ((), ())),
                         preferred_element_type=_F32)
    m_out[...] = mE * _col(ev, ones11)


def _scatter_body(msg, e, dst, hn0, hn1, hn2, hn3, ss,
                  acc, sacc, data_v, idx_v, e_v, zb2, zb1, sems, esem):
    c = lax.axis_index("c")
    s = lax.axis_index("s")
    hns = (hn0, hn1, hn2, hn3)
    zeros16 = jnp.zeros((16,), _F32)

    def zrow(i, carry):
        zb2[i, :] = zeros16
        return carry

    lax.fori_loop(0, ZR2, zrow, 0)

    def zrow1(i, carry):
        zb1[pl.ds(i * 16, 16)] = zeros16
        return carry

    lax.fori_loop(0, ZW // 16, zrow1, 0)

    def start_loads(i, b, g):
        eb = s * EPT + i * KC
        pltpu.async_copy(dst.at[pl.ds(eb, KC)], idx_v.at[b], sems.at[b, 0])
        pltpu.async_copy(msg.at[pl.ds(eb, KC), pl.ds(g * 16, 16)],
                         data_v.at[b], sems.at[b, 1])
        if g == 0:
            pltpu.async_copy(e.at[pl.ds(eb, KC)], e_v.at[b], sems.at[b, 2])

    def wait_loads(i, b, g):
        eb = s * EPT + i * KC
        pltpu.make_async_copy(dst.at[pl.ds(eb, KC)], idx_v.at[b],
                              sems.at[b, 0]).wait()
        pltpu.make_async_copy(msg.at[pl.ds(eb, KC), pl.ds(g * 16, 16)],
                              data_v.at[b], sems.at[b, 1]).wait()
        if g == 0:
            pltpu.make_async_copy(e.at[pl.ds(eb, KC)], e_v.at[b],
                                  sems.at[b, 2]).wait()

    NCH = EPT // KC

    for core in range(NC):
        @pl.when(c == core)
        def _(core=core):
            for r in range(2):
                g = core * 2 + r

                def zacc(j, carry):
                    pltpu.sync_copy(zb2, acc.at[pl.ds(s * SPT + j * ZR2, ZR2)])
                    return carry

                lax.fori_loop(0, SPT // ZR2, zacc, 0)
                if g == 0:
                    def zsacc(j, carry):
                        pltpu.sync_copy(
                            zb1, sacc.at[pl.ds(s * SPT + j * ZW, ZW)])
                        return carry
                    lax.fori_loop(0, SPT // ZW, zsacc, 0)
                plsc.subcore_barrier()

                start_loads(0, 0, g)

                def chunk(i, carry, g=g):
                    b = lax.rem(i, 2)
                    nb = lax.rem(i + 1, 2)

                    @pl.when(i + 1 < NCH)
                    def _():
                        start_loads(i + 1, nb, g)

                    wait_loads(i, b, g)
                    pltpu.async_copy(data_v.at[b], acc.at[idx_v.at[b]],
                                     esem).wait()
                    if g == 0:
                        pltpu.async_copy(e_v.at[b], sacc.at[idx_v.at[b]],
                                         esem).wait()
                    return carry

                lax.fori_loop(0, NCH, chunk, 0)
                plsc.subcore_barrier()
                pltpu.sync_copy(acc.at[pl.ds(s * SPT, SPT)],
                                hns[g].at[pl.ds(s * SPT, SPT)])
                if g == 0:
                    pltpu.sync_copy(sacc.at[pl.ds(s * SPT, SPT)],
                                    ss.at[pl.ds(s * SPT, SPT)])
                plsc.subcore_barrier()


def _nodesum_body(hn0, hn1, hn2, hn3, ssum, na, W2T, gbr,
                  Wr, Wz, Wn, Vr, Vz, Vn, bir, biz, bin_, bhr, bhz, bhn,
                  w1r, b1r, ones11, sumx):
    i = pl.program_id(0)
    Ws = (Wr, Wz, Wn, Vr, Vz, Vn, bir, biz, bin_, bhr, bhz, bhn)
    xn = _xnew(hn0, hn1, hn2, hn3, ssum, na, W2T, gbr, Ws, w1r, b1r, ones11)
    xn = jnp.where(_mask_col(i), xn, 0.0)

    @pl.when(i == 0)
    def _():
        sumx[...] = jnp.zeros_like(sumx)

    sumx[...] += jnp.sum(xn, axis=0, keepdims=True)


def _readout_body(hn0, hn1, hn2, hn3, ssum, na, W2T, gbr,
                  Wr, Wz, Wn, Vr, Vz, Vn, bir, biz, bin_, bhr, bhz, bhn,
                  w1r, b1r, ones11, sumx, molWT, attsc, attdr, num, den):
    i = pl.program_id(0)
    Ws = (Wr, Wz, Wn, Vr, Vz, Vn, bir, biz, bin_, bhr, bhz, bhn)
    xn = _xnew(hn0, hn1, hn2, hn3, ssum, na, W2T, gbr, Ws, w1r, b1r, ones11)
    xs = jnp.dot(xn, molWT[...], preferred_element_type=_F32)
    out0 = jnp.maximum(sumx[...], 0.0)
    xd = jnp.dot(out0, molWT[...], preferred_element_type=_F32)
    const = jnp.sum(xd * attdr[...])
    ap = _lrelu(jnp.dot(xs, attsc[...], preferred_element_type=_F32) + const)
    w = jnp.where(_mask_col(i), jnp.exp(ap), 0.0)

    @pl.when(i == 0)
    def _():
        num[...] = jnp.zeros_like(num)
        den[...] = jnp.zeros_like(den)

    num[...] += jnp.sum(xs * w, axis=0, keepdims=True)
    den[...] += jnp.sum(w).reshape(1, 1)


def _final_body(sumx, num, den, molbr, mWihT, mWhhT, mbihr, mbhhr,
                l2WT, l2br, out):
    out0 = jnp.maximum(sumx[...], 0.0)
    g = _elu(num[...] / den[0, 0] + molbr[...])
    o = jnp.maximum(_gru_new(g, out0, mWihT, mWhhT, mbihr, mbhhr), 0.0)
    out[...] = jnp.dot(o, l2WT[...], preferred_element_type=_F32) + l2br[...]


def _wspec(shape):
    return pl.BlockSpec(shape, lambda *_: tuple(0 for _ in shape))


_N3 = pl.BlockSpec((1, 1, BN), lambda i: (i, 0, 0))
_E3 = pl.BlockSpec((1, 1, BE), lambda i: (i, 0, 0))


def kernel(node_attr, edge_index, edge_attr, lin1_W, lin1_b, gate_lin1_W,
           gate_lin2_W, gate_att_l, gate_att_r, gate_bias, gru_W_ih, gru_W_hh,
           gru_b_ih, gru_b_hh, mol_W, mol_att_src, mol_att_dst, mol_bias,
           mgru_W_ih, mgru_W_hh, mgru_b_ih, mgru_b_hh, lin2_W, lin2_b):
    f32 = _F32
    na = jnp.concatenate([node_attr.reshape(N),
                          jnp.zeros((NPAD - N,), jnp.float32)])
    izeros = jnp.zeros((PADE,), jnp.int32)
    src = jnp.concatenate([edge_index[0], izeros])
    dst = jnp.concatenate([edge_index[1],
                           N + (jnp.arange(PADE, dtype=jnp.int32)
                                % (NPAD - N))])
    t_pad = jnp.concatenate([edge_attr.reshape(E),
                             jnp.zeros((PADE,), jnp.float32)])

    w1r = lin1_W.reshape(1, H)
    b1r = lin1_b.reshape(1, H)
    w1c = lin1_W.reshape(H, 1)
    b1c = lin1_b.reshape(H, 1)
    W1a = gate_lin1_W[:, :H]
    wec = gate_lin1_W[:, H].reshape(H, 1)
    attlr = gate_att_l.reshape(1, H)
    attrc = gate_att_r.reshape(H, 1)
    gbr = gate_bias.reshape(1, H)
    W2T = gate_lin2_W.T
    Wr, Wz, Wn = (gru_W_ih[i * H:(i + 1) * H].T for i in range(3))
    Vr, Vz, Vn = (gru_W_hh[i * H:(i + 1) * H].T for i in range(3))
    bir, biz, bin_ = (gru_b_ih[i * H:(i + 1) * H].reshape(1, H)
                      for i in range(3))
    bhr, bhz, bhn = (gru_b_hh[i * H:(i + 1) * H].reshape(1, H)
                     for i in range(3))
    molWT = mol_W.T
    attsc = mol_att_src.reshape(H, 1)
    attdr = mol_att_dst.reshape(1, H)
    molbr = mol_bias.reshape(1, H)
    mWihT = mgru_W_ih.T
    mWhhT = mgru_W_hh.T
    mbihr = mgru_b_ih.reshape(1, 3 * H)
    mbhhr = mgru_b_hh.reshape(1, 3 * H)
    l2WT = lin2_W.T
    l2br = lin2_b.reshape(1, H)
    eye = jnp.eye(H, dtype=f32)
    ones11 = jnp.ones((1, 1), f32)

    u3 = pl.pallas_call(
        _pre_body,
        grid=(GNP,),
        in_specs=[_N3, _wspec((H, 1)), _wspec((H, 1)), _wspec((H, 1))],
        out_specs=_N3,
        out_shape=jax.ShapeDtypeStruct((GNP, 1, BN), f32),
    )(na.reshape(GNP, 1, BN), w1c, b1c, attrc)

    asrc, udst = pl.kernel(
        _gather_body,
        out_type=(jax.ShapeDtypeStruct((EP,), f32),
                  jax.ShapeDtypeStruct((EP,), f32)),
        mesh=plsc.VectorSubcoreMesh(core_axis_name="c", subcore_axis_name="s"),
        scratch_types=(pltpu.VMEM((KA,), jnp.int32),
                       pltpu.VMEM((KA,), f32),
                       pltpu.SemaphoreType.DMA),
    )(na, u3.reshape(NPAD), src, dst)

    e3, msg = pl.pallas_call(
        _edge_body,
        grid=(GEP,),
        in_specs=[_E3, _E3, _E3,
                  _wspec((H, 1)), _wspec((H, 1)), _wspec((H, H)),
                  _wspec((H, 1)), _wspec((1, H)), _wspec((H, H)),
                  _wspec((1, 1))],
        out_specs=[_E3, pl.BlockSpec((BE, H), lambda i: (i, 0))],
        out_shape=[jax.ShapeDtypeStruct((GEP, 1, BE), f32),
                   jax.ShapeDtypeStruct((EP, H), f32)],
    )(asrc.reshape(GEP, 1, BE), udst.reshape(GEP, 1, BE),
      t_pad.reshape(GEP, 1, BE), w1c, b1c, W1a, wec, attlr, eye, ones11)

    hn0, hn1, hn2, hn3, ssum_pad = pl.kernel(
        _scatter_body,
        out_type=tuple([jax.ShapeDtypeStruct((NPAD, 16), f32)] * 4) +
                 (jax.ShapeDtypeStruct((NPAD,), f32),),
        mesh=plsc.VectorSubcoreMesh(core_axis_name="c", subcore_axis_name="s"),
        compiler_params=pltpu.CompilerParams(use_tc_tiling_on_sc=False),
        scratch_types=(pltpu.VMEM_SHARED((NPAD, 16), f32),
                       pltpu.VMEM_SHARED((NPAD,), f32),
                       pltpu.VMEM((2, KC, 16), f32),
                       pltpu.VMEM((2, KC), jnp.int32),
                       pltpu.VMEM((2, KC), f32),
                       pltpu.VMEM((ZR2, 16), f32),
                       pltpu.VMEM((ZW,), f32),
                       pltpu.SemaphoreType.DMA((2, 3)),
                       pltpu.SemaphoreType.DMA),
    )(msg, e3.reshape(EP), dst)

    hnspec = [pl.BlockSpec((BN, 16), lambda i: (i, 0))] * 4
    node_ins = [hn0, hn1, hn2, hn3, ssum_pad.reshape(GNP, 1, BN),
                na.reshape(GNP, 1, BN), W2T, gbr,
                Wr, Wz, Wn, Vr, Vz, Vn, bir, biz, bin_, bhr, bhz, bhn,
                w1r, b1r, ones11]
    node_specs = hnspec + [_N3, _N3, _wspec((H, H)), _wspec((1, H))] + \
                 [_wspec((H, H))] * 6 + [_wspec((1, H))] * 6 + \
                 [_wspec((1, H)), _wspec((1, H)), _wspec((1, 1))]

    sumx = pl.pallas_call(
        _nodesum_body,
        grid=(GNP,),
        in_specs=node_specs,
        out_specs=pl.BlockSpec((1, H), lambda i: (0, 0)),
        out_shape=jax.ShapeDtypeStruct((1, H), f32),
    )(*node_ins)

    num, den = pl.pallas_call(
        _readout_body,
        grid=(GNP,),
        in_specs=node_specs + [_wspec((1, H)), _wspec((H, H)),
                               _wspec((H, 1)), _wspec((1, H))],
        out_specs=[pl.BlockSpec((1, H), lambda i: (0, 0)),
                   pl.BlockSpec((1, 1), lambda i: (0, 0))],
        out_shape=[jax.ShapeDtypeStruct((1, H), f32),
                   jax.ShapeDtypeStruct((1, 1), f32)],
    )(*node_ins, sumx, molWT, attsc, attdr)

    res = pl.pallas_call(
        _final_body,
        in_specs=[_wspec((1, H)), _wspec((1, H)), _wspec((1, 1)),
                  _wspec((1, H)), _wspec((H, 3 * H)), _wspec((H, 3 * H)),
                  _wspec((1, 3 * H)), _wspec((1, 3 * H)),
                  _wspec((H, H)), _wspec((1, H))],
        out_specs=_wspec((1, H)),
        out_shape=jax.ShapeDtypeStruct((1, H), f32),
    )(sumx, num, den, molbr, mWihT, mWhhT, mbihr, mbhhr, l2WT, l2br)
    return res
